# baseline (speedup 1.0000x reference)
import jax
import jax.numpy as jnp
from jax.experimental import pallas as pl
from jax.experimental.pallas import tpu as pltpu

_GAMMA = 0.1
_BM = 80


def _fused_kernel(x_ref, w1_ref, sc0_ref, b0_ref, dk0_ref, db0_ref,
                  w2_ref, sc1_ref, b1_ref, dk1_ref, db1_ref,
                  adj_ref, adjk_ref, out_ref,
                  xw_ref, s_ref, d_ref, x1_ref, *, nb):
    i = pl.program_id(0)

    @pl.when(i == 0)
    def _prologue0():
        x = x_ref[...]
        xw_ref[...] = jnp.dot(x, w1_ref[...], preferred_element_type=jnp.float32)
        s_ref[...] = jax.nn.sigmoid(
            jnp.dot(x, sc0_ref[...], preferred_element_type=jnp.float32)
            + b0_ref[0, 0])
        d_ref[...] = (jnp.dot(x, dk0_ref[...], preferred_element_type=jnp.float32)
                      + db0_ref[0, 0])

    @pl.when(i == nb)
    def _prologue1():
        x1 = x1_ref[...]
        xw_ref[...] = jnp.dot(x1, w2_ref[...], preferred_element_type=jnp.float32)
        s_ref[...] = jax.nn.sigmoid(
            jnp.dot(x1, sc1_ref[...], preferred_element_type=jnp.float32)
            + b1_ref[0, 0])
        d_ref[...] = (jnp.dot(x1, dk1_ref[...], preferred_element_type=jnp.float32)
                      + db1_ref[0, 0])

    rows = pl.ds(jax.lax.rem(i, nb) * _BM, _BM)
    s = s_ref[rows, :]
    p = jnp.dot(adj_ref[...], xw_ref[...], preferred_element_type=jnp.float32)
    q = jnp.dot(adjk_ref[...], xw_ref[...], preferred_element_type=jnp.float32)
    res = (s * p + (1.0 - s) * q
           + _GAMMA * d_ref[rows, :] * xw_ref[rows, :])

    @pl.when(i < nb)
    def _save_x1():
        x1_ref[rows, :] = res

    @pl.when(i >= nb)
    def _save_z():
        out_ref[...] = res


def kernel(x, adj, adj_knn, W1, W2, scores0, bias0, Dk0, Dbias0,
           scores1, bias1, Dk1, Dbias1):
    n = adj.shape[0]
    h = W1.shape[1]
    nb = n // _BM
    import functools
    body = functools.partial(_fused_kernel, nb=nb)
    blk = pl.BlockSpec((_BM, n), lambda i: (jax.lax.rem(i, nb), 0))
    out_blk = pl.BlockSpec((_BM, h), lambda i: (jnp.maximum(i - nb, 0), 0))
    full = lambda r, c: pl.BlockSpec((r, c), lambda i: (0, 0))
    return pl.pallas_call(
        body,
        grid=(2 * nb,),
        in_specs=[full(n, x.shape[1]),
                  full(*W1.shape), full(*scores0.shape), full(1, 1),
                  full(*Dk0.shape), full(1, 1),
                  full(*W2.shape), full(*scores1.shape), full(1, 1),
                  full(*Dk1.shape), full(1, 1),
                  blk, blk],
        out_specs=out_blk,
        out_shape=jax.ShapeDtypeStruct((n, h), jnp.float32),
        scratch_shapes=[
            pltpu.VMEM((n, h), jnp.float32),
            pltpu.VMEM((n, 1), jnp.float32),
            pltpu.VMEM((n, 1), jnp.float32),
            pltpu.VMEM((n, h), jnp.float32),
        ],
        compiler_params=pltpu.CompilerParams(
            dimension_semantics=("arbitrary",),
            vmem_limit_bytes=100 * 1024 * 1024),
    )(x, W1, scores0, bias0.reshape(1, 1), Dk0, Dbias0.reshape(1, 1),
      W2, scores1, bias1.reshape(1, 1), Dk1, Dbias1.reshape(1, 1),
      adj, adj_knn)


# speedup vs baseline: 1.0834x; 1.0834x over previous
import jax
import jax.numpy as jnp
from jax.experimental import pallas as pl
from jax.experimental.pallas import tpu as pltpu

_GAMMA = 0.1
_BM = 200


def _fused_kernel(x_ref, w1_ref, sc0_ref, b0_ref, dk0_ref, db0_ref,
                  w2_ref, sc1_ref, b1_ref, dk1_ref, db1_ref,
                  adj_ref, adjk_ref, out_ref,
                  xw_ref, s_ref, d_ref, x1_ref, *, nb):
    i = pl.program_id(0)

    @pl.when(i == 0)
    def _prologue0():
        x = x_ref[...]
        xw_ref[...] = jnp.dot(x, w1_ref[...], preferred_element_type=jnp.float32)
        s_ref[...] = jax.nn.sigmoid(
            jnp.dot(x, sc0_ref[...], preferred_element_type=jnp.float32)
            + b0_ref[0, 0])
        d_ref[...] = (jnp.dot(x, dk0_ref[...], preferred_element_type=jnp.float32)
                      + db0_ref[0, 0])

    @pl.when(i == nb)
    def _prologue1():
        x1 = x1_ref[...]
        xw_ref[...] = jnp.dot(x1, w2_ref[...], preferred_element_type=jnp.float32)
        s_ref[...] = jax.nn.sigmoid(
            jnp.dot(x1, sc1_ref[...], preferred_element_type=jnp.float32)
            + b1_ref[0, 0])
        d_ref[...] = (jnp.dot(x1, dk1_ref[...], preferred_element_type=jnp.float32)
                      + db1_ref[0, 0])

    rows = pl.ds(jax.lax.rem(i, nb) * _BM, _BM)
    s = s_ref[rows, :]
    p = jnp.dot(adj_ref[...], xw_ref[...], preferred_element_type=jnp.float32)
    q = jnp.dot(adjk_ref[...], xw_ref[...], preferred_element_type=jnp.float32)
    res = (s * p + (1.0 - s) * q
           + _GAMMA * d_ref[rows, :] * xw_ref[rows, :])

    @pl.when(i < nb)
    def _save_x1():
        x1_ref[rows, :] = res

    @pl.when(i >= nb)
    def _save_z():
        out_ref[...] = res


def kernel(x, adj, adj_knn, W1, W2, scores0, bias0, Dk0, Dbias0,
           scores1, bias1, Dk1, Dbias1):
    n = adj.shape[0]
    h = W1.shape[1]
    nb = n // _BM
    import functools
    body = functools.partial(_fused_kernel, nb=nb)
    blk = pl.BlockSpec((_BM, n), lambda i: (jax.lax.rem(i, nb), 0))
    out_blk = pl.BlockSpec((_BM, h), lambda i: (jnp.maximum(i - nb, 0), 0))
    full = lambda r, c: pl.BlockSpec((r, c), lambda i: (0, 0))
    return pl.pallas_call(
        body,
        grid=(2 * nb,),
        in_specs=[full(n, x.shape[1]),
                  full(*W1.shape), full(*scores0.shape), full(1, 1),
                  full(*Dk0.shape), full(1, 1),
                  full(*W2.shape), full(*scores1.shape), full(1, 1),
                  full(*Dk1.shape), full(1, 1),
                  blk, blk],
        out_specs=out_blk,
        out_shape=jax.ShapeDtypeStruct((n, h), jnp.float32),
        scratch_shapes=[
            pltpu.VMEM((n, h), jnp.float32),
            pltpu.VMEM((n, 1), jnp.float32),
            pltpu.VMEM((n, 1), jnp.float32),
            pltpu.VMEM((n, h), jnp.float32),
        ],
        compiler_params=pltpu.CompilerParams(
            dimension_semantics=("arbitrary",),
            vmem_limit_bytes=100 * 1024 * 1024),
    )(x, W1, scores0, bias0.reshape(1, 1), Dk0, Dbias0.reshape(1, 1),
      W2, scores1, bias1.reshape(1, 1), Dk1, Dbias1.reshape(1, 1),
      adj, adj_knn)


# speedup vs baseline: 1.1074x; 1.0222x over previous
import functools

import jax
import jax.numpy as jnp
from jax.experimental import pallas as pl
from jax.experimental.pallas import tpu as pltpu

_GAMMA = 0.1
_BM = 200


def _fused_kernel(x_ref, c0_ref, b0_ref, c1_ref, b1_ref,
                  adj_ref, adjk_ref, out_ref,
                  scr_ref, x1_ref, *, nb, h):
    i = pl.program_id(0)

    @pl.when(i == 0)
    def _prologue0():
        scr_ref[...] = (jnp.dot(x_ref[...], c0_ref[...],
                                preferred_element_type=jnp.float32)
                        + b0_ref[...])

    @pl.when(i == nb)
    def _prologue1():
        scr_ref[...] = (jnp.dot(x1_ref[...], c1_ref[...],
                                preferred_element_type=jnp.float32)
                        + b1_ref[...])

    rows = pl.ds(jax.lax.rem(i, nb) * _BM, _BM)
    xw = scr_ref[:, 0:h]
    s = jax.nn.sigmoid(scr_ref[rows, h:h + 1])
    d = scr_ref[rows, h + 1:h + 2]
    p = jnp.dot(adj_ref[...], xw, preferred_element_type=jnp.float32)
    q = jnp.dot(adjk_ref[...], xw, preferred_element_type=jnp.float32)
    res = q + s * (p - q) + (_GAMMA * d) * scr_ref[rows, 0:h]

    @pl.when(i < nb)
    def _save_x1():
        x1_ref[rows, :] = res

    @pl.when(i >= nb)
    def _save_z():
        out_ref[...] = res


def kernel(x, adj, adj_knn, W1, W2, scores0, bias0, Dk0, Dbias0,
           scores1, bias1, Dk1, Dbias1):
    n = adj.shape[0]
    h = W1.shape[1]
    nb = n // _BM
    zeros_h = jnp.zeros((1, h), jnp.float32)
    c0 = jnp.concatenate([W1, scores0, Dk0], axis=1)
    b0 = jnp.concatenate([zeros_h, bias0.reshape(1, 1),
                          Dbias0.reshape(1, 1)], axis=1)
    c1 = jnp.concatenate([W2, scores1, Dk1], axis=1)
    b1 = jnp.concatenate([zeros_h, bias1.reshape(1, 1),
                          Dbias1.reshape(1, 1)], axis=1)
    body = functools.partial(_fused_kernel, nb=nb, h=h)
    blk = pl.BlockSpec((_BM, n), lambda i: (jax.lax.rem(i, nb), 0))
    out_blk = pl.BlockSpec((_BM, h), lambda i: (jnp.maximum(i - nb, 0), 0))
    full = lambda r, c: pl.BlockSpec((r, c), lambda i: (0, 0))
    return pl.pallas_call(
        body,
        grid=(2 * nb,),
        in_specs=[full(n, x.shape[1]),
                  full(*c0.shape), full(*b0.shape),
                  full(*c1.shape), full(*b1.shape),
                  blk, blk],
        out_specs=out_blk,
        out_shape=jax.ShapeDtypeStruct((n, h), jnp.float32),
        scratch_shapes=[
            pltpu.VMEM((n, h + 2), jnp.float32),
            pltpu.VMEM((n, h), jnp.float32),
        ],
        compiler_params=pltpu.CompilerParams(
            dimension_semantics=("arbitrary",),
            vmem_limit_bytes=100 * 1024 * 1024),
    )(x, c0, b0, c1, b1, adj, adj_knn)
